# baseline (speedup 1.0000x reference)
import functools

import numpy as np
import jax
import jax.numpy as jnp
from jax import lax
from jax.experimental import pallas as pl
from jax.experimental.pallas import tpu as pltpu
from jax.experimental.pallas import tpu_sc as plsc

D_M = 64
L_SEQ = 200
ENC_BASE = 10000.0


def _pos_table_np():
    pos = np.arange(L_SEQ, dtype=np.float32)[:, None]
    i = np.arange(D_M // 2, dtype=np.float32)[None, :]
    denoms = pos / np.power(np.float32(ENC_BASE), 2.0 * i / np.float32(D_M))
    mat = np.zeros((L_SEQ, 128), dtype=np.float32)
    mat[:, 0:D_M:2] = np.sin(denoms)
    mat[:, 1:D_M:2] = np.cos(denoms)
    return mat


@functools.lru_cache(maxsize=None)
def _make_kernels(V, B, L):
    info = plsc.get_sparse_core_info()
    NC, NS, LANES = info.num_cores, info.num_subcores, info.num_lanes
    NW = NC * NS

    n_blocks_full = V // 128
    tail_cols = V - n_blocks_full * 128
    n_sub = n_blocks_full // NW + 2
    if n_sub % 2:
        n_sub += 1
    NBJ = B // 128

    mesh = plsc.VectorSubcoreMesh(core_axis_name="c", subcore_axis_name="s")

    @functools.partial(
        pl.kernel,
        mesh=mesh,
        compiler_params=pltpu.CompilerParams(needs_layout_passes=False),
        out_type=jax.ShapeDtypeStruct((V // 2, 2 * D_M), jnp.float32),
        scratch_types=[
            pltpu.VMEM((2, D_M, 128), jnp.float32),
            pltpu.VMEM((2, D_M, 128), jnp.float32),
            pltpu.VMEM((D_M, 129), jnp.float32),
            pltpu.SemaphoreType.DMA,
            pltpu.SemaphoreType.DMA,
            pltpu.SemaphoreType.DMA,
            pltpu.SemaphoreType.DMA,
        ],
    )
    def k1(tT_hbm, tailp_hbm, scr_hbm, in_v, out_v, stg_v, is0, is1, os0, os1):
        wid = lax.axis_index("s") * NC + lax.axis_index("c")
        ivec = lax.iota(jnp.int32, LANES)
        rowidx = [ivec + 16 * (q % 4) for q in range(4)]

        def start_in(t, buf, sem):
            j = wid + NW * t
            pltpu.async_copy(
                tT_hbm.at[:, pl.ds(pl.multiple_of(128 * j, 128), 128)], in_v.at[buf], sem
            )

        def transpose_block(dst, r):
            for q in range(8):
                col = jnp.broadcast_to(2 * r + q // 4, (LANES,))
                val = plsc.load_gather(stg_v, [rowidx[q % 4], col])
                dst[r, pl.ds(16 * q, 16)] = val

        def sub_step(t, b, isem, osem, isem_n, osem_n):
            j = wid + NW * t
            live = j < n_blocks_full

            @pl.when((wid + NW * (t + 1)) < n_blocks_full)
            def _():
                start_in(t + 1, 1 - b, isem_n)

            @pl.when(live)
            def _():
                pltpu.make_async_copy(
                    tT_hbm.at[:, pl.ds(pl.multiple_of(128 * j, 128), 128)], in_v.at[b], isem
                ).wait()

            @pl.when(jnp.logical_and(live, t >= 2))
            def _():
                pltpu.make_async_copy(
                    out_v.at[b],
                    scr_hbm.at[pl.ds(pl.multiple_of(64 * (j - 2 * NW), 64), 64)],
                    osem,
                ).wait()

            @pl.when(live)
            def _():
                def stage(r, carry):
                    for g in range(8):
                        sl = pl.ds(16 * g, 16)
                        stg_v[r, sl] = in_v[b, r, sl]
                    return carry

                lax.fori_loop(0, D_M, stage, 0)

                def body(r, carry):
                    transpose_block(out_v.at[b], r)
                    return carry

                lax.fori_loop(0, 64, body, 0)
                pltpu.async_copy(
                    out_v.at[b], scr_hbm.at[pl.ds(pl.multiple_of(64 * j, 64), 64)], osem
                )

        start_in(0, 0, is0)

        def pair_body(i, carry):
            sub_step(2 * i, 0, is0, os0, is1, os1)
            sub_step(2 * i + 1, 1, is1, os1, is0, os0)
            return carry

        lax.fori_loop(0, n_sub // 2, pair_body, 0)

        pltpu.make_async_copy(out_v.at[0], scr_hbm.at[pl.ds(0, 64)], os0).wait()
        pltpu.make_async_copy(out_v.at[1], scr_hbm.at[pl.ds(0, 64)], os1).wait()

        @pl.when(wid == n_blocks_full % NW)
        def _():
            pltpu.sync_copy(tailp_hbm, in_v.at[0, pl.ds(0, tail_cols // 2)])
            pltpu.sync_copy(
                in_v.at[0, pl.ds(0, tail_cols // 2)],
                scr_hbm.at[pl.ds((V - tail_cols) // 2, tail_cols // 2)],
            )

    @functools.partial(
        pl.kernel,
        mesh=mesh,
        compiler_params=pltpu.CompilerParams(needs_layout_passes=False),
        out_type=jax.ShapeDtypeStruct(
            (L, D_M // 8, NBJ, 8, 128), jnp.float32
        ),
        scratch_types=[
            pltpu.VMEM((L, 128), jnp.int32),
            pltpu.VMEM((L_SEQ, 128), jnp.float32),
            pltpu.VMEM((2, 128), jnp.int32),
            pltpu.VMEM((2, 128, 128), jnp.float32),
            pltpu.VMEM((D_M // 8, 8, 128), jnp.float32),
            pltpu.VMEM((128, 129), jnp.float32),
            pltpu.SemaphoreType.DMA,
            pltpu.SemaphoreType.DMA,
            pltpu.SemaphoreType.DMA,
            pltpu.SemaphoreType.DMA,
        ],
    )
    def k2(scr_hbm, xT_hbm, pos_hbm, out_hbm, xs_v, pos_v, qb_v, rows_v,
           ob_v, stg_v, gs0, gs1, ws0, ws1):
        wid = lax.axis_index("s") * NC + lax.axis_index("c")
        pltpu.sync_copy(pos_hbm, pos_v)
        pltpu.sync_copy(xT_hbm.at[:, pl.ds(pl.multiple_of(128 * wid, 128), 128)], xs_v)
        ivec = lax.iota(jnp.int32, LANES)

        def fill_q(l, buf):
            for g in range(8):
                iv = xs_v[l, pl.ds(16 * g, 16)]
                qb_v[buf, pl.ds(16 * g, 16)] = lax.shift_right_logical(iv, 1)

        def start_gather(l, buf, sem):
            pltpu.async_copy(scr_hbm.at[qb_v.at[buf]], rows_v.at[buf], sem)

        def sub_step(l, b, gsem, wsem, gsem_n):
            pltpu.make_async_copy(
                scr_hbm.at[pl.ds(0, 128)], rows_v.at[b], gsem
            ).wait()

            @pl.when(l + 1 < L)
            def _():
                fill_q(l + 1, 1 - b)
                start_gather(l + 1, 1 - b, gsem_n)

            pass

            colbase = []
            for g in range(8):
                iv = xs_v[l, pl.ds(16 * g, 16)]
                colbase.append(jnp.bitwise_and(iv, 1) * D_M)

            lvec = jnp.broadcast_to(l, (LANES,))

            def stage(r, carry):
                for g in range(8):
                    sl = pl.ds(16 * g, 16)
                    stg_v[r, sl] = rows_v[b, r, sl]
                return carry

            lax.fori_loop(0, 128, stage, 0)

            def d_body(dhi, carry):
                for dlo in range(8):
                    d = dhi * 8 + dlo
                    dvec = jnp.broadcast_to(d, (LANES,))
                    posd = plsc.load_gather(pos_v, [lvec, dvec])
                    for g in range(8):
                        val = plsc.load_gather(
                            stg_v, [ivec + 16 * g, colbase[g] + d]
                        )
                        ob_v[dhi, dlo, pl.ds(16 * g, 16)] = val + posd
                return carry

            @pl.when(l >= 1)
            def _():
                pltpu.make_async_copy(
                    ob_v, out_hbm.at[l - 1, :, wid], ws0
                ).wait()

            lax.fori_loop(0, D_M // 8, d_body, 0)
            pltpu.async_copy(ob_v, out_hbm.at[l, :, wid], ws0)

        fill_q(0, 0)
        start_gather(0, 0, gs0)

        def pair_body(i, carry):
            sub_step(2 * i, 0, gs0, ws0, gs1)
            sub_step(2 * i + 1, 1, gs1, ws1, gs0)
            return carry

        lax.fori_loop(0, L // 2, pair_body, 0)

        pltpu.make_async_copy(
            ob_v, out_hbm.at[L - 1, :, wid], ws0
        ).wait()

    return k1, k2


def kernel(x, table):
    B, L = x.shape
    V = table.shape[0]
    tT = table.T
    xT = x.T
    pos = jnp.asarray(_pos_table_np())
    k1, k2 = _make_kernels(V, B, L)
    tailp = table[V - V % 128:, :].reshape(-1, 2 * D_M)
    scratch = k1(tT, tailp)
    out5 = k2(scratch, xT, pos)
    out = jnp.transpose(out5, (2, 4, 0, 1, 3)).reshape(B, L, D_M)
    return out


# speedup vs baseline: 3.1085x; 3.1085x over previous
import functools

import numpy as np
import jax
import jax.numpy as jnp
from jax import lax
from jax.experimental import pallas as pl
from jax.experimental.pallas import tpu as pltpu
from jax.experimental.pallas import tpu_sc as plsc

D_M = 64
L_SEQ = 200
ENC_BASE = 10000.0
G = 128
CH = 512
NG = CH // G


def _pos_table_np():
    pos = np.arange(L_SEQ, dtype=np.float32)[:, None]
    i = np.arange(D_M // 2, dtype=np.float32)[None, :]
    denoms = pos / np.power(np.float32(ENC_BASE), 2.0 * i / np.float32(D_M))
    mat = np.zeros((L_SEQ, D_M), dtype=np.float32)
    mat[:, 0::2] = np.sin(denoms)
    mat[:, 1::2] = np.cos(denoms)
    return mat


@functools.lru_cache(maxsize=None)
def _make_sc_kernel(BL, V):
    info = plsc.get_sparse_core_info()
    NC, NS, LANES = info.num_cores, info.num_subcores, info.num_lanes
    NW = NC * NS
    rows_per_w = BL // NW
    n_chunks = rows_per_w // CH
    n_idx_rows = rows_per_w // G

    mesh = plsc.VectorSubcoreMesh(core_axis_name="c", subcore_axis_name="s")

    @functools.partial(
        pl.kernel,
        mesh=mesh,
        compiler_params=pltpu.CompilerParams(use_tc_tiling_on_sc=False),
        out_type=jax.ShapeDtypeStruct((BL, D_M), jnp.float32),
        scratch_types=[
            pltpu.VMEM((n_idx_rows, G), jnp.int32),
            pltpu.VMEM((L_SEQ, D_M), jnp.float32),
            pltpu.VMEM((2, CH, D_M), jnp.float32),
            pltpu.SemaphoreType.DMA,
            pltpu.SemaphoreType.DMA,
            pltpu.SemaphoreType.DMA,
            pltpu.SemaphoreType.DMA,
        ],
    )
    def k(x_hbm, pos_hbm, table_hbm, out_hbm, idx_v, pos_v, rows_v,
          gsem0, gsem1, wsem0, wsem1):
        wid = lax.axis_index("s") * NC + lax.axis_index("c")
        pltpu.sync_copy(pos_hbm, pos_v)
        pltpu.sync_copy(x_hbm.at[pl.ds(wid * n_idx_rows, n_idx_rows)], idx_v)
        wbase = wid * rows_per_w

        def start_gathers(c, b, gsem):
            for j in range(NG):
                pltpu.async_copy(
                    table_hbm.at[idx_v.at[c * NG + j]],
                    rows_v.at[b, pl.ds(j * G, G)],
                    gsem,
                )

        def sub_step(c, b, gsem, wsem, gsem_n, wsem_n):
            pltpu.make_async_copy(
                out_hbm.at[pl.ds(wbase + c * CH, CH)], rows_v.at[b], gsem
            ).wait()

            @pl.when(c >= 1)
            def _():
                pltpu.make_async_copy(
                    rows_v.at[1 - b],
                    out_hbm.at[pl.ds(wbase + (c - 1) * CH, CH)],
                    wsem_n,
                ).wait()

            @pl.when(c + 1 < n_chunks)
            def _():
                start_gathers(c + 1, 1 - b, gsem_n)

            def add_pair(i, l):
                l1 = lax.select(l + 1 == L_SEQ, 0, l + 1)
                vals = []
                for r, lv in ((2 * i, l), (2 * i + 1, l1)):
                    for kk in range(D_M // LANES):
                        s = pl.ds(kk * LANES, LANES)
                        vals.append((r, kk, rows_v[b, r, s] + pos_v[lv, s]))
                for r, kk, v in vals:
                    rows_v[b, r, pl.ds(kk * LANES, LANES)] = v
                l = l + 2
                return lax.select(l >= L_SEQ, l - L_SEQ, l)

            l0 = lax.rem(c * CH, L_SEQ)
            lax.fori_loop(0, CH // 2, add_pair, l0)

            pltpu.async_copy(
                rows_v.at[b], out_hbm.at[pl.ds(wbase + c * CH, CH)], wsem
            )

        start_gathers(0, 0, gsem0)

        def pair_body(i, carry):
            sub_step(2 * i, 0, gsem0, wsem0, gsem1, wsem1)
            sub_step(2 * i + 1, 1, gsem1, wsem1, gsem0, wsem0)
            return carry

        lax.fori_loop(0, n_chunks // 2, pair_body, 0)

        pltpu.make_async_copy(
            rows_v.at[1],
            out_hbm.at[pl.ds(wbase + (n_chunks - 1) * CH, CH)],
            wsem1,
        ).wait()

    return k


def kernel(x, table):
    B, L = x.shape
    BL = B * L
    x2 = x.reshape(BL // G, G)
    pos = jnp.asarray(_pos_table_np())
    k = _make_sc_kernel(BL, table.shape[0])
    out = k(x2, pos, table)
    return out.reshape(B, L, D_M)


# speedup vs baseline: 3.1254x; 1.0054x over previous
import functools

import numpy as np
import jax
import jax.numpy as jnp
from jax import lax
from jax.experimental import pallas as pl
from jax.experimental.pallas import tpu as pltpu
from jax.experimental.pallas import tpu_sc as plsc

D_M = 64
L_SEQ = 200
ENC_BASE = 10000.0
G = 128
CH = 512
NG = CH // G


def _pos_table_np():
    pos = np.arange(L_SEQ, dtype=np.float32)[:, None]
    i = np.arange(D_M // 2, dtype=np.float32)[None, :]
    denoms = pos / np.power(np.float32(ENC_BASE), 2.0 * i / np.float32(D_M))
    mat = np.zeros((L_SEQ, D_M), dtype=np.float32)
    mat[:, 0::2] = np.sin(denoms)
    mat[:, 1::2] = np.cos(denoms)
    return mat


@functools.lru_cache(maxsize=None)
def _make_sc_kernel(BL, V):
    info = plsc.get_sparse_core_info()
    NC, NS, LANES = info.num_cores, info.num_subcores, info.num_lanes
    NW = NC * NS
    rows_per_w = BL // NW
    n_chunks = rows_per_w // CH
    n_idx_rows = rows_per_w // G

    mesh = plsc.VectorSubcoreMesh(core_axis_name="c", subcore_axis_name="s")

    @functools.partial(
        pl.kernel,
        mesh=mesh,
        compiler_params=pltpu.CompilerParams(use_tc_tiling_on_sc=False),
        out_type=jax.ShapeDtypeStruct((BL, D_M), jnp.float32),
        scratch_types=[
            pltpu.VMEM((n_idx_rows, G), jnp.int32),
            pltpu.VMEM((L_SEQ, D_M), jnp.float32),
            pltpu.VMEM((2, CH, D_M), jnp.float32),
            pltpu.SemaphoreType.DMA,
            pltpu.SemaphoreType.DMA,
            pltpu.SemaphoreType.DMA,
            pltpu.SemaphoreType.DMA,
        ],
    )
    def k(x_hbm, pos_hbm, table_hbm, out_hbm, idx_v, pos_v, rows_v,
          gsem0, gsem1, wsem0, wsem1):
        wid = lax.axis_index("s") * NC + lax.axis_index("c")
        pltpu.sync_copy(pos_hbm, pos_v)
        pltpu.sync_copy(x_hbm.at[pl.ds(wid * n_idx_rows, n_idx_rows)], idx_v)
        wbase = wid * rows_per_w

        def start_gathers(c, b, gsem):
            for j in range(NG):
                pltpu.async_copy(
                    table_hbm.at[idx_v.at[c * NG + j]],
                    rows_v.at[b, pl.ds(j * G, G)],
                    gsem,
                )

        def sub_step(c, b, gsem, wsem, gsem_n, wsem_n):
            pltpu.make_async_copy(
                out_hbm.at[pl.ds(wbase + c * CH, CH)], rows_v.at[b], gsem
            ).wait()

            @pl.when(c >= 1)
            def _():
                pltpu.make_async_copy(
                    rows_v.at[1 - b],
                    out_hbm.at[pl.ds(wbase + (c - 1) * CH, CH)],
                    wsem_n,
                ).wait()

            @pl.when(c + 1 < n_chunks)
            def _():
                start_gathers(c + 1, 1 - b, gsem_n)

            def add_quad(i, l):
                lvs = [l]
                for _ in range(3):
                    nl = lvs[-1] + 1
                    lvs.append(lax.select(nl == L_SEQ, 0, nl))
                vals = []
                for u in range(4):
                    for kk in range(D_M // LANES):
                        s = pl.ds(kk * LANES, LANES)
                        vals.append(
                            (4 * i + u, kk,
                             rows_v[b, 4 * i + u, s] + pos_v[lvs[u], s])
                        )
                for r, kk, v in vals:
                    rows_v[b, r, pl.ds(kk * LANES, LANES)] = v
                l = l + 4
                return lax.select(l >= L_SEQ, l - L_SEQ, l)

            l0 = lax.rem(c * CH, L_SEQ)
            lax.fori_loop(0, CH // 4, add_quad, l0)

            pltpu.async_copy(
                rows_v.at[b], out_hbm.at[pl.ds(wbase + c * CH, CH)], wsem
            )

        start_gathers(0, 0, gsem0)

        def pair_body(i, carry):
            sub_step(2 * i, 0, gsem0, wsem0, gsem1, wsem1)
            sub_step(2 * i + 1, 1, gsem1, wsem1, gsem0, wsem0)
            return carry

        lax.fori_loop(0, n_chunks // 2, pair_body, 0)

        pltpu.make_async_copy(
            rows_v.at[1],
            out_hbm.at[pl.ds(wbase + (n_chunks - 1) * CH, CH)],
            wsem1,
        ).wait()

    return k


def kernel(x, table):
    B, L = x.shape
    BL = B * L
    x2 = x.reshape(BL // G, G)
    pos = jnp.asarray(_pos_table_np())
    k = _make_sc_kernel(BL, table.shape[0])
    out = k(x2, pos, table)
    return out.reshape(B, L, D_M)


# speedup vs baseline: 3.1296x; 1.0013x over previous
import functools

import numpy as np
import jax
import jax.numpy as jnp
from jax import lax
from jax.experimental import pallas as pl
from jax.experimental.pallas import tpu as pltpu
from jax.experimental.pallas import tpu_sc as plsc

D_M = 64
L_SEQ = 200
ENC_BASE = 10000.0
G = 128
CH = 640
NG = CH // G


def _pos_table_np():
    pos = np.arange(L_SEQ, dtype=np.float32)[:, None]
    i = np.arange(D_M // 2, dtype=np.float32)[None, :]
    denoms = pos / np.power(np.float32(ENC_BASE), 2.0 * i / np.float32(D_M))
    mat = np.zeros((L_SEQ, D_M), dtype=np.float32)
    mat[:, 0::2] = np.sin(denoms)
    mat[:, 1::2] = np.cos(denoms)
    return mat


@functools.lru_cache(maxsize=None)
def _make_sc_kernel(BL, V):
    info = plsc.get_sparse_core_info()
    NC, NS, LANES = info.num_cores, info.num_subcores, info.num_lanes
    NW = NC * NS
    rows_per_w = BL // NW
    n_chunks = rows_per_w // CH
    n_idx_rows = rows_per_w // G

    mesh = plsc.VectorSubcoreMesh(core_axis_name="c", subcore_axis_name="s")

    @functools.partial(
        pl.kernel,
        mesh=mesh,
        compiler_params=pltpu.CompilerParams(use_tc_tiling_on_sc=False),
        out_type=jax.ShapeDtypeStruct((BL, D_M), jnp.float32),
        scratch_types=[
            pltpu.VMEM((n_idx_rows, G), jnp.int32),
            pltpu.VMEM((L_SEQ, D_M), jnp.float32),
            pltpu.VMEM((2, CH, D_M), jnp.float32),
            pltpu.SemaphoreType.DMA,
            pltpu.SemaphoreType.DMA,
            pltpu.SemaphoreType.DMA,
            pltpu.SemaphoreType.DMA,
        ],
    )
    def k(x_hbm, pos_hbm, table_hbm, out_hbm, idx_v, pos_v, rows_v,
          gsem0, gsem1, wsem0, wsem1):
        wid = lax.axis_index("s") * NC + lax.axis_index("c")
        pltpu.sync_copy(pos_hbm, pos_v)
        pltpu.sync_copy(x_hbm.at[pl.ds(wid * n_idx_rows, n_idx_rows)], idx_v)
        wbase = wid * rows_per_w

        def start_gathers(c, b, gsem):
            for j in range(NG):
                pltpu.async_copy(
                    table_hbm.at[idx_v.at[c * NG + j]],
                    rows_v.at[b, pl.ds(j * G, G)],
                    gsem,
                )

        def sub_step(c, b, gsem, wsem, gsem_n, wsem_n):
            pltpu.make_async_copy(
                out_hbm.at[pl.ds(wbase + c * CH, CH)], rows_v.at[b], gsem
            ).wait()

            @pl.when(c >= 1)
            def _():
                pltpu.make_async_copy(
                    rows_v.at[1 - b],
                    out_hbm.at[pl.ds(wbase + (c - 1) * CH, CH)],
                    wsem_n,
                ).wait()

            @pl.when(c + 1 < n_chunks)
            def _():
                start_gathers(c + 1, 1 - b, gsem_n)

            def add_quad(i, l):
                lvs = [l]
                for _ in range(3):
                    nl = lvs[-1] + 1
                    lvs.append(lax.select(nl == L_SEQ, 0, nl))
                vals = []
                for u in range(4):
                    for kk in range(D_M // LANES):
                        s = pl.ds(kk * LANES, LANES)
                        vals.append(
                            (4 * i + u, kk,
                             rows_v[b, 4 * i + u, s] + pos_v[lvs[u], s])
                        )
                for r, kk, v in vals:
                    rows_v[b, r, pl.ds(kk * LANES, LANES)] = v
                l = l + 4
                return lax.select(l >= L_SEQ, l - L_SEQ, l)

            l0 = lax.rem(c * CH, L_SEQ)
            lax.fori_loop(0, CH // 4, add_quad, l0)

            pltpu.async_copy(
                rows_v.at[b], out_hbm.at[pl.ds(wbase + c * CH, CH)], wsem
            )

        start_gathers(0, 0, gsem0)

        def pair_body(i, carry):
            sub_step(2 * i, 0, gsem0, wsem0, gsem1, wsem1)
            sub_step(2 * i + 1, 1, gsem1, wsem1, gsem0, wsem0)
            return carry

        lax.fori_loop(0, n_chunks // 2, pair_body, 0)

        pltpu.make_async_copy(
            rows_v.at[1],
            out_hbm.at[pl.ds(wbase + (n_chunks - 1) * CH, CH)],
            wsem1,
        ).wait()

    return k


def kernel(x, table):
    B, L = x.shape
    BL = B * L
    x2 = x.reshape(BL // G, G)
    pos = jnp.asarray(_pos_table_np())
    k = _make_sc_kernel(BL, table.shape[0])
    out = k(x2, pos, table)
    return out.reshape(B, L, D_M)
